# Initial kernel scaffold; baseline (speedup 1.0000x reference)
import functools

import jax
import jax.numpy as jnp
from jax import lax
from jax.experimental import pallas as pl
from jax.experimental.pallas import tpu as pltpu
from jax.experimental.pallas import tpu_sc as plsc

N = 10000
E = 320000
D = 128

NC = 2
NS = 16
NW = NC * NS

EB = 125
NB = (E // NW) // EB
NBT = E // EB

N2 = 10240
STRIPE = N2 // NS

_mesh = plsc.VectorSubcoreMesh(
    core_axis_name="c", subcore_axis_name="s", num_cores=NC, num_subcores=NS
)


@functools.partial(
    pl.kernel,
    out_type=jax.ShapeDtypeStruct((NC, N2), jnp.float32),
    mesh=_mesh,
    scratch_types=[
        pltpu.VMEM((NB, EB), jnp.int32),
        pltpu.VMEM((NB, EB), jnp.float32),
        pltpu.VMEM((STRIPE,), jnp.float32),
        pltpu.VMEM_SHARED((N2,), jnp.float32),
    ],
)
def _deg_sc(dst2d, w2d, degp, dst_v, w_v, zbuf, acc):
    c = lax.axis_index("c")
    s = lax.axis_index("s")
    wid = s * NC + c
    pltpu.sync_copy(dst2d.at[pl.ds(wid * NB, NB)], dst_v)
    pltpu.sync_copy(w2d.at[pl.ds(wid * NB, NB)], w_v)

    def zb(i, carry):
        zbuf[pl.ds(i * 16, 16)] = jnp.zeros((16,), jnp.float32)
        return carry

    lax.fori_loop(0, STRIPE // 16, zb, None)
    pltpu.sync_copy(zbuf, acc.at[pl.ds(s * STRIPE, STRIPE)])
    plsc.subcore_barrier()

    def body(j, carry):
        pltpu.sync_copy(w_v.at[j], acc.at[dst_v.at[j]], add=True)
        return carry

    lax.fori_loop(0, NB, body, None)
    plsc.subcore_barrier()
    pltpu.sync_copy(acc.at[pl.ds(s * STRIPE, STRIPE)],
                    degp.at[c, pl.ds(s * STRIPE, STRIPE)])


@functools.partial(
    pl.kernel,
    out_type=jax.ShapeDtypeStruct((NC, N2, D), jnp.float32),
    mesh=_mesh,
    scratch_types=[
        pltpu.VMEM((NB, EB), jnp.int32),
        pltpu.VMEM((NB, EB), jnp.int32),
        pltpu.VMEM((NB, EB), jnp.float32),
        pltpu.VMEM((EB, D), jnp.float32),
        pltpu.VMEM((STRIPE // 5, D), jnp.float32),
        pltpu.VMEM_SHARED((N2, D), jnp.float32),
        pltpu.SemaphoreType.DMA,
    ],
)
def _edge_sc(xws, src2d, dst2d, w2d, part, src_v, dst_v, w_v, rows, zbuf, acc,
             sem):
    c = lax.axis_index("c")
    s = lax.axis_index("s")
    wid = s * NC + c
    pltpu.sync_copy(src2d.at[pl.ds(wid * NB, NB)], src_v)
    pltpu.sync_copy(dst2d.at[pl.ds(wid * NB, NB)], dst_v)
    pltpu.sync_copy(w2d.at[pl.ds(wid * NB, NB)], w_v)

    def zb(r, carry):
        for k in range(D // 16):
            zbuf[r, pl.ds(k * 16, 16)] = jnp.zeros((16,), jnp.float32)
        return carry

    lax.fori_loop(0, STRIPE // 5, zb, None)
    for k in range(5):
        pltpu.sync_copy(zbuf, acc.at[pl.ds(s * STRIPE + k * (STRIPE // 5),
                                           STRIPE // 5)])
    plsc.subcore_barrier()

    def body(j, carry):
        pltpu.async_copy(xws.at[src_v.at[j]], rows, sem).wait()

        def mul(r, carry2):
            w = w_v[j, r]
            for k in range(D // 16):
                sl = pl.ds(k * 16, 16)
                rows[r, sl] = rows[r, sl] * w
            return carry2

        lax.fori_loop(0, EB, mul, None)
        pltpu.sync_copy(rows, acc.at[dst_v.at[j]], add=True)
        return carry

    lax.fori_loop(0, NB, body, None)
    plsc.subcore_barrier()
    for k in range(5):
        sl = pl.ds(s * STRIPE + k * (STRIPE // 5), STRIPE // 5)
        pltpu.sync_copy(acc.at[sl], part.at[c, sl])


BLK = 1000


def _mm2_body(x_ref, w0_ref, w1_ref, h_ref, xw1_ref):
    h = jnp.dot(x_ref[...], w0_ref[...], preferred_element_type=jnp.float32)
    h_ref[...] = h
    xw1_ref[...] = jnp.dot(h, w1_ref[...], preferred_element_type=jnp.float32)


def _dinv_body(dga_ref, dgb_ref, xw1_ref, dinv_ref, dinv2_ref, xws1_ref):
    deg = dga_ref[...] + dgb_ref[...] + 1.0
    dinv = jnp.where(deg > 0, lax.rsqrt(jnp.maximum(deg, 1e-12)), 0.0)
    dinv_ref[...] = dinv
    dinv2_ref[...] = dinv * dinv
    xws1_ref[...] = dinv * xw1_ref[...]


def _mid_body(p1a_ref, p1b_ref, xw1_ref, h_ref, dinv_ref, dinv2_ref, b1_ref,
              w2_ref, xw2_ref, xws2_ref):
    dinv = dinv_ref[...]
    pre = (dinv * (p1a_ref[...] + p1b_ref[...])
           + dinv2_ref[...] * xw1_ref[...] + b1_ref[...] + h_ref[...])
    x1 = jnp.maximum(pre, 0.0)
    xw2 = jnp.dot(x1, w2_ref[...], preferred_element_type=jnp.float32)
    xw2_ref[...] = xw2
    xws2_ref[...] = dinv * xw2


def _fin_body(p2a_ref, p2b_ref, xw2_ref, h_ref, dinv_ref, dinv2_ref, b2_ref,
              out_ref):
    out_ref[...] = (dinv_ref[...] * (p2a_ref[...] + p2b_ref[...])
                    + dinv2_ref[...] * xw2_ref[...] + b2_ref[...] + h_ref[...])


def _row_spec(width=D):
    return pl.BlockSpec((BLK, width), lambda i: (i, 0))


def _full_spec(shape):
    return pl.BlockSpec(shape, lambda i: tuple(0 for _ in shape))


_GRID = N // BLK

_mm2 = pl.pallas_call(
    _mm2_body,
    grid=(_GRID,),
    in_specs=[_row_spec(), _full_spec((D, D)), _full_spec((D, D))],
    out_specs=[_row_spec(), _row_spec()],
    out_shape=[jax.ShapeDtypeStruct((N, D), jnp.float32)] * 2,
)

_dinv_k = pl.pallas_call(
    _dinv_body,
    grid=(_GRID,),
    in_specs=[_row_spec(1), _row_spec(1), _row_spec()],
    out_specs=[_row_spec(1), _row_spec(1), _row_spec()],
    out_shape=[jax.ShapeDtypeStruct((N, 1), jnp.float32),
               jax.ShapeDtypeStruct((N, 1), jnp.float32),
               jax.ShapeDtypeStruct((N, D), jnp.float32)],
)

_mid = pl.pallas_call(
    _mid_body,
    grid=(_GRID,),
    in_specs=[_row_spec(), _row_spec(), _row_spec(), _row_spec(),
              _row_spec(1), _row_spec(1), _full_spec((1, D)),
              _full_spec((D, D))],
    out_specs=[_row_spec(), _row_spec()],
    out_shape=[jax.ShapeDtypeStruct((N, D), jnp.float32)] * 2,
)

_fin = pl.pallas_call(
    _fin_body,
    grid=(_GRID,),
    in_specs=[_row_spec(), _row_spec(), _row_spec(), _row_spec(),
              _row_spec(1), _row_spec(1), _full_spec((1, D))],
    out_specs=_row_spec(),
    out_shape=jax.ShapeDtypeStruct((N, D), jnp.float32),
)


def kernel(x, edge_index, edge_attr, W0, W1, b1, W2, b2):
    src2d = edge_index[0].reshape(NBT, EB)
    dst2d = edge_index[1].reshape(NBT, EB)
    w2d = edge_attr.reshape(NBT, EB)

    degp = _deg_sc(dst2d, w2d)
    h, xw1 = _mm2(x, W0, W1)
    dinv, dinv2, xws1 = _dinv_k(degp[0, :N, None], degp[1, :N, None], xw1)
    part1 = _edge_sc(xws1, src2d, dst2d, w2d)
    xw2, xws2 = _mid(part1[0, :N], part1[1, :N], xw1, h, dinv, dinv2,
                     b1.reshape(1, D), W2)
    part2 = _edge_sc(xws2, src2d, dst2d, w2d)
    return _fin(part2[0, :N], part2[1, :N], xw2, h, dinv, dinv2,
                b2.reshape(1, D))


# speedup vs baseline: 12.2249x; 12.2249x over previous
import functools

import jax
import jax.numpy as jnp
from jax import lax
from jax.experimental import pallas as pl
from jax.experimental.pallas import tpu as pltpu
from jax.experimental.pallas import tpu_sc as plsc

N = 10000
E = 320000
D = 128

NC = 2
NS = 16
NW = NC * NS

EPT = E // NW
EB = 128
NB = 79
PAD = NB * EB - EPT

N2 = 10240
STRIPE = N2 // NS

_mesh = plsc.VectorSubcoreMesh(
    core_axis_name="c", subcore_axis_name="s", num_cores=NC, num_subcores=NS
)


@functools.partial(
    pl.kernel,
    out_type=jax.ShapeDtypeStruct((NC, 1, N2), jnp.float32),
    mesh=_mesh,
    scratch_types=[
        pltpu.VMEM((NB, EB), jnp.int32),
        pltpu.VMEM((NB, EB), jnp.float32),
        pltpu.VMEM((STRIPE,), jnp.float32),
        pltpu.VMEM_SHARED((N2,), jnp.float32),
    ],
)
def _deg_sc(dst3, w3, degp, dst_v, w_v, zbuf, acc):
    c = lax.axis_index("c")
    s = lax.axis_index("s")
    wid = s * NC + c
    pltpu.sync_copy(dst3.at[wid], dst_v)
    pltpu.sync_copy(w3.at[wid], w_v)

    def zb(i, carry):
        zbuf[pl.ds(i * 16, 16)] = jnp.zeros((16,), jnp.float32)
        return carry

    lax.fori_loop(0, STRIPE // 16, zb, None)
    pltpu.sync_copy(zbuf, acc.at[pl.ds(s * STRIPE, STRIPE)])
    plsc.subcore_barrier()

    def body(j, carry):
        pltpu.sync_copy(w_v.at[j], acc.at[dst_v.at[j]], add=True)
        return carry

    lax.fori_loop(0, NB, body, None)
    plsc.subcore_barrier()
    pltpu.sync_copy(acc.at[pl.ds(s * STRIPE, STRIPE)],
                    degp.at[c, 0, pl.ds(s * STRIPE, STRIPE)])


@functools.partial(
    pl.kernel,
    out_type=jax.ShapeDtypeStruct((NC, N2, D), jnp.float32),
    mesh=_mesh,
    scratch_types=[
        pltpu.VMEM((NB, EB), jnp.int32),
        pltpu.VMEM((NB, EB), jnp.int32),
        pltpu.VMEM((NB, EB), jnp.float32),
        pltpu.VMEM((EB, D), jnp.float32),
        pltpu.VMEM_SHARED((N2, D), jnp.float32),
        pltpu.SemaphoreType.DMA,
    ],
)
def _edge_sc(xws, src3, dst3, w3, part, src_v, dst_v, w_v, rows, acc, sem):
    c = lax.axis_index("c")
    s = lax.axis_index("s")
    wid = s * NC + c
    pltpu.sync_copy(src3.at[wid], src_v)
    pltpu.sync_copy(dst3.at[wid], dst_v)
    pltpu.sync_copy(w3.at[wid], w_v)

    def zb(r, carry):
        for k in range(D // 16):
            rows[r, pl.ds(k * 16, 16)] = jnp.zeros((16,), jnp.float32)
        return carry

    lax.fori_loop(0, EB, zb, None)
    for k in range(STRIPE // EB):
        pltpu.sync_copy(rows, acc.at[pl.ds(s * STRIPE + k * EB, EB)])
    plsc.subcore_barrier()

    def body(j, carry):
        pltpu.async_copy(xws.at[src_v.at[j]], rows, sem).wait()

        def mul(b, carry2):
            wblk = w_v[j, pl.ds(b * 16, 16)]
            for i in range(16):
                w = wblk[i]
                for k in range(D // 16):
                    sl = pl.ds(k * 16, 16)
                    rows[b * 16 + i, sl] = rows[b * 16 + i, sl] * w
            return carry2

        lax.fori_loop(0, EB // 16, mul, None)
        pltpu.sync_copy(rows, acc.at[dst_v.at[j]], add=True)
        return carry

    lax.fori_loop(0, NB, body, None)
    plsc.subcore_barrier()
    for k in range(STRIPE // EB):
        sl = pl.ds(s * STRIPE + k * EB, EB)
        pltpu.sync_copy(acc.at[sl], part.at[c, sl])


BLK = 1000


def _mm2_body(x_ref, w0_ref, w1_ref, h_ref, xw1_ref):
    h = jnp.dot(x_ref[...], w0_ref[...], preferred_element_type=jnp.float32)
    h_ref[...] = h
    xw1_ref[...] = jnp.dot(h, w1_ref[...], preferred_element_type=jnp.float32)


def _dinv_body(dga_ref, dgb_ref, xw1_ref, dinv_ref, dinv2_ref, xws1_ref):
    deg = dga_ref[...] + dgb_ref[...] + 1.0
    dinv = jnp.where(deg > 0, lax.rsqrt(jnp.maximum(deg, 1e-12)), 0.0)
    dinv_ref[...] = dinv
    dinv2_ref[...] = dinv * dinv
    xws1_ref[...] = dinv * xw1_ref[...]


def _mid_body(p1a_ref, p1b_ref, xw1_ref, h_ref, dinv_ref, dinv2_ref, b1_ref,
              w2_ref, xw2_ref, xws2_ref):
    dinv = dinv_ref[...]
    pre = (dinv * (p1a_ref[...] + p1b_ref[...])
           + dinv2_ref[...] * xw1_ref[...] + b1_ref[...] + h_ref[...])
    x1 = jnp.maximum(pre, 0.0)
    xw2 = jnp.dot(x1, w2_ref[...], preferred_element_type=jnp.float32)
    xw2_ref[...] = xw2
    xws2_ref[...] = dinv * xw2


def _fin_body(p2a_ref, p2b_ref, xw2_ref, h_ref, dinv_ref, dinv2_ref, b2_ref,
              out_ref):
    out_ref[...] = (dinv_ref[...] * (p2a_ref[...] + p2b_ref[...])
                    + dinv2_ref[...] * xw2_ref[...] + b2_ref[...] + h_ref[...])


def _row_spec(width=D):
    return pl.BlockSpec((BLK, width), lambda i: (i, 0))


def _full_spec(shape):
    return pl.BlockSpec(shape, lambda i: tuple(0 for _ in shape))


_GRID = N // BLK

_mm2 = pl.pallas_call(
    _mm2_body,
    grid=(_GRID,),
    in_specs=[_row_spec(), _full_spec((D, D)), _full_spec((D, D))],
    out_specs=[_row_spec(), _row_spec()],
    out_shape=[jax.ShapeDtypeStruct((N, D), jnp.float32)] * 2,
)

_dinv_k = pl.pallas_call(
    _dinv_body,
    grid=(_GRID,),
    in_specs=[_row_spec(1), _row_spec(1), _row_spec()],
    out_specs=[_row_spec(1), _row_spec(1), _row_spec()],
    out_shape=[jax.ShapeDtypeStruct((N, 1), jnp.float32),
               jax.ShapeDtypeStruct((N, 1), jnp.float32),
               jax.ShapeDtypeStruct((N, D), jnp.float32)],
)

_mid = pl.pallas_call(
    _mid_body,
    grid=(_GRID,),
    in_specs=[_row_spec(), _row_spec(), _row_spec(), _row_spec(),
              _row_spec(1), _row_spec(1), _full_spec((1, D)),
              _full_spec((D, D))],
    out_specs=[_row_spec(), _row_spec()],
    out_shape=[jax.ShapeDtypeStruct((N, D), jnp.float32)] * 2,
)

_fin = pl.pallas_call(
    _fin_body,
    grid=(_GRID,),
    in_specs=[_row_spec(), _row_spec(), _row_spec(), _row_spec(),
              _row_spec(1), _row_spec(1), _full_spec((1, D))],
    out_specs=_row_spec(),
    out_shape=jax.ShapeDtypeStruct((N, D), jnp.float32),
)


def kernel(x, edge_index, edge_attr, W0, W1, b1, W2, b2):
    zi = jnp.zeros((NW, PAD), jnp.int32)
    zf = jnp.zeros((NW, PAD), jnp.float32)
    src3 = jnp.concatenate(
        [edge_index[0].reshape(NW, EPT), zi], axis=1).reshape(NW, NB, EB)
    dst3 = jnp.concatenate(
        [edge_index[1].reshape(NW, EPT), zi], axis=1).reshape(NW, NB, EB)
    w3 = jnp.concatenate(
        [edge_attr.reshape(NW, EPT), zf], axis=1).reshape(NW, NB, EB)

    degp = _deg_sc(dst3, w3)
    h, xw1 = _mm2(x, W0, W1)
    dinv, dinv2, xws1 = _dinv_k(degp[0, 0, :N, None], degp[1, 0, :N, None],
                                xw1)
    part1 = _edge_sc(xws1, src3, dst3, w3)
    xw2, xws2 = _mid(part1[0, :N], part1[1, :N], xw1, h, dinv, dinv2,
                     b1.reshape(1, D), W2)
    part2 = _edge_sc(xws2, src3, dst3, w3)
    return _fin(part2[0, :N], part2[1, :N], xw2, h, dinv, dinv2,
                b2.reshape(1, D))
